# baseline (speedup 1.0000x reference)
import functools

import jax
import jax.numpy as jnp
from jax import lax
from jax.experimental import pallas as pl
from jax.experimental.pallas import tpu as pltpu
from jax.experimental.pallas import tpu_sc as plsc

N_CLASSES = 1000000
EMBED_DIM = 16
BATCH = 16384

NUM_CORES = 2
NUM_SUBCORES = 16
NW = NUM_CORES * NUM_SUBCORES
CHUNK = 128
BPW = BATCH // NW
CPW = BPW // CHUNK

_mesh = plsc.VectorSubcoreMesh(core_axis_name="c", subcore_axis_name="s")


@functools.partial(
    pl.kernel,
    mesh=_mesh,
    out_type=jax.ShapeDtypeStruct((BATCH, 1, EMBED_DIM), jnp.float32),
    scratch_types=[
        pltpu.VMEM((BPW,), jnp.int32),
        pltpu.VMEM((BPW, EMBED_DIM), jnp.float32),
        pltpu.SemaphoreType.DMA,
    ],
    compiler_params=pltpu.CompilerParams(use_tc_tiling_on_sc=False),
)
def _embed_lookup(idx_hbm, table_hbm, out_hbm, idx_v, rows_v, sem):
    wid = lax.axis_index("s") * NUM_CORES + lax.axis_index("c")
    base = wid * BPW
    pltpu.sync_copy(idx_hbm.at[pl.ds(base, BPW)], idx_v)
    copies = [
        pltpu.async_copy(
            table_hbm.at[idx_v.at[pl.ds(j * CHUNK, CHUNK)]],
            rows_v.at[pl.ds(j * CHUNK, CHUNK)],
            sem,
        )
        for j in range(CPW)
    ]
    for c in copies:
        c.wait()
    pltpu.sync_copy(rows_v, out_hbm.at[pl.ds(base, BPW), 0])


def kernel(cond, table):
    return _embed_lookup(cond, table)


# speedup vs baseline: 5.4857x; 5.4857x over previous
import functools

import jax
import jax.numpy as jnp
from jax import lax
from jax.experimental import pallas as pl
from jax.experimental.pallas import tpu as pltpu
from jax.experimental.pallas import tpu_sc as plsc

N_CLASSES = 1000000
EMBED_DIM = 16
BATCH = 16384

NUM_CORES = 2
NUM_SUBCORES = 16
NW = NUM_CORES * NUM_SUBCORES
BPW = BATCH // NW
LANES = 16
NGRP = BPW // LANES
LAST_TILE = 7811
TAIL0 = 999936

_mesh = plsc.VectorSubcoreMesh(core_axis_name="c", subcore_axis_name="s")


@functools.partial(
    pl.kernel,
    mesh=_mesh,
    out_type=jax.ShapeDtypeStruct((EMBED_DIM, BATCH), jnp.float32),
    scratch_types=[
        pltpu.VMEM((BPW,), jnp.int32),
        pltpu.VMEM((LANES, EMBED_DIM, 128), jnp.float32),
        pltpu.VMEM((EMBED_DIM, 64), jnp.float32),
        pltpu.VMEM((EMBED_DIM, BPW), jnp.float32),
        pltpu.SemaphoreType.DMA,
    ],
    compiler_params=pltpu.CompilerParams(needs_layout_passes=False),
)
def _embed_lookup(idx_hbm, tableT_hbm, out_hbm, idx_v, blk_v,
                  tail_v, rows_v, sem):
    sid = lax.axis_index("s")
    wid = sid * NUM_CORES + lax.axis_index("c")
    base = wid * BPW
    pltpu.sync_copy(idx_hbm.at[pl.ds(base, BPW)], idx_v)
    pltpu.sync_copy(tableT_hbm.at[:, pl.ds(TAIL0, 64)], tail_v)
    lanevec = jax.lax.iota(jnp.int32, LANES)

    def grp_body(g, _):
        rv = idx_v[pl.ds(g * LANES, LANES)]
        kc = jnp.clip(rv >> 7, 0, LAST_TILE)
        copies = []
        for i in range(LANES):
            k = jax.lax.reduce_max(
                jnp.where(lanevec == i, kc, 0), axes=(0,)
            )
            col = pl.multiple_of(k * 128, 128)
            copies.append(
                pltpu.async_copy(
                    tableT_hbm.at[:, pl.ds(col, 128)], blk_v.at[i], sem
                )
            )
        for cp in copies:
            cp.wait()

        rvec = idx_v[pl.ds(g * LANES, LANES)]
        kvec = rvec >> 7
        lane = rvec & 127
        is_tail = kvec > LAST_TILE
        lane_tail = jnp.clip(rvec - TAIL0, 0, 63)
        blkidx = jax.lax.iota(jnp.int32, LANES)
        for c in range(EMBED_DIM):
            cvec = jnp.full((LANES,), c, jnp.int32)
            main = plsc.load_gather(blk_v, [blkidx, cvec, lane])
            tailv = plsc.load_gather(tail_v, [cvec, lane_tail])
            rows_v[c, pl.ds(g * LANES, LANES)] = jnp.where(is_tail, tailv, main)
        return 0

    lax.fori_loop(0, NGRP, grp_body, 0)
    pltpu.sync_copy(rows_v, out_hbm.at[:, pl.ds(base, BPW)])


def kernel(cond, table):
    out = _embed_lookup(cond, table.T)
    return out.T.reshape(BATCH, 1, EMBED_DIM)
